# Initial kernel scaffold; baseline (speedup 1.0000x reference)
import jax
import jax.numpy as jnp
from jax import lax
from jax.experimental import pallas as pl
from jax.experimental.pallas import tpu as pltpu
from jax.experimental.pallas import tpu_sc as plsc

N_NODES = 10000
D = 128
N_EDGES = 320000

NC = 2
NS = 16
NW = NC * NS
CH = 128
NCHUNK = N_EDGES // CH
RPT = N_NODES // NS

_mesh = plsc.VectorSubcoreMesh(core_axis_name="c", subcore_axis_name="s")


def _deg_hist_body(col_hbm, zeros_hbm, ones_hbm, out_hbm, idx_v, ones_v, acc_sh):
    cid = lax.axis_index("c")
    sid = lax.axis_index("s")
    wid = sid * NC + cid
    base = sid * RPT
    pltpu.sync_copy(ones_hbm, ones_v)
    pltpu.sync_copy(zeros_hbm.at[pl.ds(base, RPT)], acc_sh.at[pl.ds(base, RPT)])
    plsc.subcore_barrier()

    @pl.loop(wid, NCHUNK, step=NW)
    def _(c):
        pltpu.sync_copy(col_hbm.at[pl.ds(c * CH, CH)], idx_v)
        pltpu.sync_copy(ones_v, acc_sh.at[idx_v], add=True)

    plsc.subcore_barrier()
    pltpu.sync_copy(acc_sh.at[pl.ds(base, RPT)],
                    out_hbm.at[cid, pl.ds(base, RPT)])


def _agg_body(y_hbm, row_hbm, col_hbm, zeros_hbm, out_hbm,
              rowi_v, coli_v, rows_v, acc_sh):
    cid = lax.axis_index("c")
    sid = lax.axis_index("s")
    wid = sid * NC + cid
    base = sid * RPT

    @pl.when(cid == 0)
    def _():
        pltpu.sync_copy(y_hbm.at[pl.ds(base, RPT)], acc_sh.at[pl.ds(base, RPT)])

    @pl.when(cid == 1)
    def _():
        pltpu.sync_copy(zeros_hbm.at[pl.ds(base, RPT)],
                        acc_sh.at[pl.ds(base, RPT)])

    plsc.subcore_barrier()

    @pl.loop(wid, NCHUNK, step=NW)
    def _(c):
        pltpu.sync_copy(row_hbm.at[pl.ds(c * CH, CH)], rowi_v)
        pltpu.sync_copy(col_hbm.at[pl.ds(c * CH, CH)], coli_v)
        pltpu.sync_copy(y_hbm.at[rowi_v], rows_v)
        pltpu.sync_copy(rows_v, acc_sh.at[coli_v], add=True)

    plsc.subcore_barrier()
    pltpu.sync_copy(acc_sh.at[pl.ds(base, RPT)],
                    out_hbm.at[cid, pl.ds(base, RPT)])


def _linear_body(x_ref, w_ref, slab_ref, y_ref):
    deg = slab_ref[0][:, 0:1] + slab_ref[1][:, 0:1] + 1.0
    dis = lax.rsqrt(deg)
    y_ref[...] = dis * jnp.dot(x_ref[...], w_ref[...],
                               preferred_element_type=jnp.float32)


def _epilogue_body(agg_ref, x_ref, b_ref, slab_ref, out_ref):
    deg = slab_ref[0][:, 0:1] + slab_ref[1][:, 0:1] + 1.0
    dis = lax.rsqrt(deg)
    s = agg_ref[0] + agg_ref[1]
    out_ref[...] = jnp.maximum(dis * s + b_ref[...], 0.0) + x_ref[...]


def kernel(x, edge_index, W, b):
    row = edge_index[0].astype(jnp.int32)
    col = edge_index[1].astype(jnp.int32)
    zeros16 = jnp.zeros((N_NODES, 16), jnp.float32)
    zeros128 = jnp.zeros((N_NODES, D), jnp.float32)
    ones16 = jnp.ones((CH, 16), jnp.float32)

    deg_hist = pl.kernel(
        _deg_hist_body,
        out_type=jax.ShapeDtypeStruct((NC, N_NODES, 16), jnp.float32),
        mesh=_mesh,
        scratch_types=[
            pltpu.VMEM((CH,), jnp.int32),
            pltpu.VMEM((CH, 16), jnp.float32),
            pltpu.VMEM_SHARED((N_NODES, 16), jnp.float32),
        ],
    )
    slabs = deg_hist(col, zeros16, ones16)

    R = 1000
    y = pl.pallas_call(
        _linear_body,
        grid=(N_NODES // R,),
        in_specs=[
            pl.BlockSpec((R, D), lambda i: (i, 0)),
            pl.BlockSpec((D, D), lambda i: (0, 0)),
            pl.BlockSpec((NC, R, 16), lambda i: (0, i, 0)),
        ],
        out_specs=pl.BlockSpec((R, D), lambda i: (i, 0)),
        out_shape=jax.ShapeDtypeStruct((N_NODES, D), jnp.float32),
    )(x, W, slabs)

    agg_call = pl.kernel(
        _agg_body,
        out_type=jax.ShapeDtypeStruct((NC, N_NODES, D), jnp.float32),
        mesh=_mesh,
        scratch_types=[
            pltpu.VMEM((CH,), jnp.int32),
            pltpu.VMEM((CH,), jnp.int32),
            pltpu.VMEM((CH, D), jnp.float32),
            pltpu.VMEM_SHARED((N_NODES, D), jnp.float32),
        ],
    )
    agg = agg_call(y, row, col, zeros128)

    out = pl.pallas_call(
        _epilogue_body,
        grid=(N_NODES // R,),
        in_specs=[
            pl.BlockSpec((NC, R, D), lambda i: (0, i, 0)),
            pl.BlockSpec((R, D), lambda i: (i, 0)),
            pl.BlockSpec((1, D), lambda i: (0, 0)),
            pl.BlockSpec((NC, R, 16), lambda i: (0, i, 0)),
        ],
        out_specs=pl.BlockSpec((R, D), lambda i: (i, 0)),
        out_shape=jax.ShapeDtypeStruct((N_NODES, D), jnp.float32),
    )(agg, x, b.reshape(1, D), slabs)

    return (out, edge_index)


# speedup vs baseline: 19.6155x; 19.6155x over previous
import dataclasses

import jax
import jax.numpy as jnp
from jax import lax
from jax.experimental import pallas as pl
from jax.experimental.pallas import tpu as pltpu
from jax.experimental.pallas import tpu_sc as plsc

N_NODES = 10000
D = 128
N_EDGES = 320000

NC = 2
NS = 16
NW = NC * NS
CH = 128
NCHUNK = N_EDGES // CH
SPAN = 624
TAIL_BASE = NS * SPAN
TAIL = N_NODES - TAIL_BASE

_mesh = plsc.VectorSubcoreMesh(core_axis_name="c", subcore_axis_name="s")

_sc_params = pltpu.CompilerParams()
if "needs_layout_passes" in pltpu.CompilerParams.__dataclass_fields__:
    _sc_params = dataclasses.replace(_sc_params, needs_layout_passes=False)


def _span_copy(sid, src, dst):
    base = sid * SPAN
    pltpu.sync_copy(src.at[pl.ds(base, SPAN)], dst.at[pl.ds(base, SPAN)])

    @pl.when(sid == NS - 1)
    def _():
        pltpu.sync_copy(src.at[pl.ds(TAIL_BASE, TAIL)],
                        dst.at[pl.ds(TAIL_BASE, TAIL)])


def _deg_hist_body(col_hbm, out_hbm, idx_v, deg_v):
    cid = lax.axis_index("c")
    sid = lax.axis_index("s")
    wid = sid * NC + cid

    @pl.loop(0, N_NODES // 16)
    def _(r):
        deg_v[pl.ds(r * 16, 16)] = jnp.zeros((16,), jnp.float32)

    ones = jnp.ones((16,), jnp.float32)

    @pl.loop(wid, NCHUNK, step=NW)
    def _(c):
        pltpu.sync_copy(col_hbm.at[pl.ds(c * CH, CH)], idx_v)
        for j in range(CH // 16):
            idx16 = idx_v[pl.ds(j * 16, 16)]
            plsc.addupdate_scatter(deg_v, [idx16], ones)

    pltpu.sync_copy(deg_v, out_hbm.at[pl.ds(wid * N_NODES, N_NODES)])


def _agg_body(y_hbm, row_hbm, col_hbm, zeros_hbm, out_hbm,
              rowi_v, coli_v, rows_v, acc_sh):
    cid = lax.axis_index("c")
    sid = lax.axis_index("s")
    wid = sid * NC + cid

    @pl.when(cid == 0)
    def _():
        _span_copy(sid, y_hbm, acc_sh)

    @pl.when(cid == 1)
    def _():
        _span_copy(sid, zeros_hbm, acc_sh)

    plsc.subcore_barrier()

    @pl.loop(wid, NCHUNK, step=NW)
    def _(c):
        pltpu.sync_copy(row_hbm.at[pl.ds(c * CH, CH)], rowi_v)
        pltpu.sync_copy(col_hbm.at[pl.ds(c * CH, CH)], coli_v)
        pltpu.sync_copy(y_hbm.at[rowi_v], rows_v)
        pltpu.sync_copy(rows_v, acc_sh.at[coli_v], add=True)

    plsc.subcore_barrier()
    _span_copy(sid, acc_sh, out_hbm.at[cid])


def _dis_body(parts_ref, dis_ref):
    deg = jnp.sum(parts_ref[...], axis=0, keepdims=True) + 1.0
    dis_ref[...] = jnp.transpose(lax.rsqrt(deg), (1, 0))


def _linear_body(x_ref, w_ref, dis_ref, y_ref):
    y_ref[...] = dis_ref[...] * jnp.dot(x_ref[...], w_ref[...],
                                        preferred_element_type=jnp.float32)


def _epilogue_body(agg_ref, x_ref, b_ref, dis_ref, out_ref):
    s = agg_ref[0] + agg_ref[1]
    out_ref[...] = jnp.maximum(dis_ref[...] * s + b_ref[...], 0.0) + x_ref[...]


def kernel(x, edge_index, W, b):
    row = edge_index[0].astype(jnp.int32)
    col = edge_index[1].astype(jnp.int32)
    zeros128 = jnp.zeros((N_NODES, D), jnp.float32)

    deg_hist = pl.kernel(
        _deg_hist_body,
        out_type=jax.ShapeDtypeStruct((NW * N_NODES,), jnp.float32),
        mesh=_mesh,
        compiler_params=_sc_params,
        scratch_types=[
            pltpu.VMEM((CH,), jnp.int32),
            pltpu.VMEM((N_NODES,), jnp.float32),
        ],
    )
    deg_parts = deg_hist(col).reshape(NW, N_NODES)

    dis = pl.pallas_call(
        _dis_body,
        in_specs=[pl.BlockSpec((NW, N_NODES), lambda: (0, 0))],
        out_specs=pl.BlockSpec((N_NODES, 1), lambda: (0, 0)),
        out_shape=jax.ShapeDtypeStruct((N_NODES, 1), jnp.float32),
    )(deg_parts)

    R = 1000
    y = pl.pallas_call(
        _linear_body,
        grid=(N_NODES // R,),
        in_specs=[
            pl.BlockSpec((R, D), lambda i: (i, 0)),
            pl.BlockSpec((D, D), lambda i: (0, 0)),
            pl.BlockSpec((R, 1), lambda i: (i, 0)),
        ],
        out_specs=pl.BlockSpec((R, D), lambda i: (i, 0)),
        out_shape=jax.ShapeDtypeStruct((N_NODES, D), jnp.float32),
    )(x, W, dis)

    agg_call = pl.kernel(
        _agg_body,
        out_type=jax.ShapeDtypeStruct((NC, N_NODES, D), jnp.float32),
        mesh=_mesh,
        scratch_types=[
            pltpu.VMEM((CH,), jnp.int32),
            pltpu.VMEM((CH,), jnp.int32),
            pltpu.VMEM((CH, D), jnp.float32),
            pltpu.VMEM_SHARED((N_NODES, D), jnp.float32),
        ],
    )
    agg = agg_call(y, row, col, zeros128)

    out = pl.pallas_call(
        _epilogue_body,
        grid=(N_NODES // R,),
        in_specs=[
            pl.BlockSpec((NC, R, D), lambda i: (0, i, 0)),
            pl.BlockSpec((R, D), lambda i: (i, 0)),
            pl.BlockSpec((1, D), lambda i: (0, 0)),
            pl.BlockSpec((R, 1), lambda i: (i, 0)),
        ],
        out_specs=pl.BlockSpec((R, D), lambda i: (i, 0)),
        out_shape=jax.ShapeDtypeStruct((N_NODES, D), jnp.float32),
    )(agg, x, b.reshape(1, D), dis)

    return (out, edge_index)
